# Initial kernel scaffold; baseline (speedup 1.0000x reference)
import jax
import jax.numpy as jnp
from jax.experimental import pallas as pl


def kernel(x, num_atoms, radius, atom_properties_tensor, node_standardization_tensor, graph_standardization_tensor):
    raise NotImplementedError("write your pallas kernel here")


# speedup vs baseline: 5.5846x; 5.5846x over previous
import functools

import jax
import jax.numpy as jnp
from jax import lax
from jax.experimental import pallas as pl
from jax.experimental.pallas import tpu as pltpu
from jax.experimental.pallas import tpu_sc as plsc

N_NODES = 2097152
N_GRAPHS = 16384
N_ELEM = 100
NF = 6
MF = 2

NC, NS, L = 2, 16, 16
NW = NC * NS
NT = N_NODES // NW
CH = 2048
NCHUNK = NT // CH
GT = N_GRAPHS // NW

_mesh = plsc.VectorSubcoreMesh(
    core_axis_name="c", subcore_axis_name="s", num_cores=NC, num_subcores=NS
)


@functools.partial(
    pl.kernel,
    out_type=(
        jax.ShapeDtypeStruct((N_NODES * NF,), jnp.float32),
        jax.ShapeDtypeStruct((N_GRAPHS * MF,), jnp.float32),
    ),
    mesh=_mesh,
    compiler_params=pltpu.CompilerParams(needs_layout_passes=False),
    scratch_types=[
        pltpu.VMEM((NT,), jnp.int32),
        pltpu.VMEM((CH * NF,), jnp.float32),
        pltpu.VMEM((CH * NF,), jnp.float32),
        pltpu.VMEM((N_ELEM * NF,), jnp.float32),
        pltpu.VMEM((L,), jnp.float32),
        pltpu.VMEM((L,), jnp.float32),
        pltpu.VMEM((GT,), jnp.float32),
        pltpu.VMEM((GT,), jnp.float32),
        pltpu.VMEM((GT * MF,), jnp.float32),
        pltpu.SemaphoreType.DMA,
        pltpu.SemaphoreType.DMA,
    ],
)
def _featurize(x_hbm, na_hbm, rad_hbm, tab_hbm, nstd_hbm, gstd_hbm,
               node_out, mol_out,
               x_v, out_a, out_b, tab_v, nstd_v, gstd_v,
               na_v, rad_v, molo_v, sem_a, sem_b):
    wid = lax.axis_index("s") * NC + lax.axis_index("c")
    nbase = wid * NT
    gbase = wid * GT

    pltpu.sync_copy(x_hbm.at[pl.ds(nbase, NT)], x_v)
    pltpu.sync_copy(tab_hbm, tab_v)
    pltpu.sync_copy(nstd_hbm, nstd_v.at[pl.ds(0, NF * 2)])
    pltpu.sync_copy(gstd_hbm, gstd_v.at[pl.ds(0, MF * 2)])
    pltpu.sync_copy(na_hbm.at[pl.ds(gbase, GT)], na_v)
    pltpu.sync_copy(rad_hbm.at[pl.ds(gbase, GT)], rad_v)

    iota = lax.iota(jnp.int32, L)
    i6 = iota * NF
    i2 = iota * MF

    nv = nstd_v[pl.ds(0, L)]
    gv = gstd_v[pl.ds(0, L)]
    rnv = 1.0 / nv
    rgv = 1.0 / gv
    means = [jnp.full((L,), nv[2 * j]) for j in range(NF)]
    rstds = [jnp.full((L,), rnv[2 * j + 1]) for j in range(NF)]

    m_na = jnp.full((L,), gv[0])
    rs_na = jnp.full((L,), rgv[1])
    m_r = jnp.full((L,), gv[2])
    rs_r = jnp.full((L,), rgv[3])

    def mol_body(g, carry):
        na = (na_v[pl.ds(g * L, L)] - m_na) * rs_na
        rd = (rad_v[pl.ds(g * L, L)] - m_r) * rs_r
        plsc.store_scatter(molo_v, [i2 + g * (L * MF)], na)
        plsc.store_scatter(molo_v, [i2 + (g * (L * MF) + 1)], rd)
        return carry

    lax.fori_loop(0, GT // L, mol_body, 0)
    pltpu.sync_copy(molo_v, mol_out.at[pl.ds(gbase * MF, GT * MF)])

    def make_chunk_body(ch, buf):
        def g_body(g, carry):
            xv6 = x_v[pl.ds(ch * CH + g * L, L)] * NF
            for j in range(NF):
                vals = plsc.load_gather(tab_v, [xv6 + j] if j else [xv6])
                vals = (vals - means[j]) * rstds[j]
                plsc.store_scatter(buf, [i6 + (g * (L * NF) + j)], vals)
            return carry
        return g_body

    bufs = (out_a, out_b)
    sems = (sem_a, sem_b)
    pending = [None, None]
    for ch in range(NCHUNK):
        b = ch % 2
        if pending[b] is not None:
            pending[b].wait()
        lax.fori_loop(0, CH // L, make_chunk_body(ch, bufs[b]), 0)
        dst = node_out.at[pl.ds((nbase + ch * CH) * NF, CH * NF)]
        pending[b] = pltpu.async_copy(bufs[b], dst, sems[b])
    pending[0].wait()
    pending[1].wait()


def kernel(x, num_atoms, radius, atom_properties_tensor,
           node_standardization_tensor, graph_standardization_tensor):
    node_flat, mol_flat = _featurize(
        x, num_atoms, radius,
        atom_properties_tensor.reshape(-1),
        node_standardization_tensor.reshape(-1),
        graph_standardization_tensor.reshape(-1))
    return node_flat.reshape(N_NODES, NF), mol_flat.reshape(N_GRAPHS, MF)


# speedup vs baseline: 6.1857x; 1.1076x over previous
import functools

import jax
import jax.numpy as jnp
from jax import lax
from jax.experimental import pallas as pl
from jax.experimental.pallas import tpu as pltpu
from jax.experimental.pallas import tpu_sc as plsc

N_NODES = 2097152
N_GRAPHS = 16384
N_ELEM = 100
NF = 6
MF = 2

NC, NS, L = 2, 16, 16
NW = NC * NS
NT = N_NODES // NW
CH = 4096
NCHUNK = NT // CH
GT = N_GRAPHS // NW

_mesh = plsc.VectorSubcoreMesh(
    core_axis_name="c", subcore_axis_name="s", num_cores=NC, num_subcores=NS
)


@functools.partial(
    pl.kernel,
    out_type=(
        jax.ShapeDtypeStruct((N_NODES * NF,), jnp.float32),
        jax.ShapeDtypeStruct((N_GRAPHS * MF,), jnp.float32),
    ),
    mesh=_mesh,
    compiler_params=pltpu.CompilerParams(needs_layout_passes=False),
    scratch_types=[
        pltpu.VMEM((NT,), jnp.int32),
        pltpu.VMEM((CH * NF,), jnp.float32),
        pltpu.VMEM((CH * NF,), jnp.float32),
        pltpu.VMEM((N_ELEM * NF,), jnp.float32),
        pltpu.VMEM((L,), jnp.float32),
        pltpu.VMEM((L,), jnp.float32),
        pltpu.VMEM((GT,), jnp.float32),
        pltpu.VMEM((GT,), jnp.float32),
        pltpu.VMEM((GT * MF,), jnp.float32),
        pltpu.SemaphoreType.DMA,
        pltpu.SemaphoreType.DMA,
    ],
)
def _featurize(x_hbm, na_hbm, rad_hbm, tab_hbm, nstd_hbm, gstd_hbm,
               node_out, mol_out,
               x_v, out_a, out_b, tab_v, nstd_v, gstd_v,
               na_v, rad_v, molo_v, sem_a, sem_b):
    wid = lax.axis_index("s") * NC + lax.axis_index("c")
    nbase = wid * NT
    gbase = wid * GT

    pltpu.sync_copy(x_hbm.at[pl.ds(nbase, NT)], x_v)
    pltpu.sync_copy(tab_hbm, tab_v)
    pltpu.sync_copy(nstd_hbm, nstd_v.at[pl.ds(0, NF * 2)])
    pltpu.sync_copy(gstd_hbm, gstd_v.at[pl.ds(0, MF * 2)])
    pltpu.sync_copy(na_hbm.at[pl.ds(gbase, GT)], na_v)
    pltpu.sync_copy(rad_hbm.at[pl.ds(gbase, GT)], rad_v)

    iota = lax.iota(jnp.int32, L)
    i6 = iota * NF
    i2 = iota * MF

    nv = nstd_v[pl.ds(0, L)]
    gv = gstd_v[pl.ds(0, L)]
    rnv = 1.0 / nv
    rgv = 1.0 / gv
    means = [jnp.full((L,), nv[2 * j]) for j in range(NF)]
    rstds = [jnp.full((L,), rnv[2 * j + 1]) for j in range(NF)]

    m_na = jnp.full((L,), gv[0])
    rs_na = jnp.full((L,), rgv[1])
    m_r = jnp.full((L,), gv[2])
    rs_r = jnp.full((L,), rgv[3])

    def mol_body(g, carry):
        na = (na_v[pl.ds(g * L, L)] - m_na) * rs_na
        rd = (rad_v[pl.ds(g * L, L)] - m_r) * rs_r
        plsc.store_scatter(molo_v, [i2 + g * (L * MF)], na)
        plsc.store_scatter(molo_v, [i2 + (g * (L * MF) + 1)], rd)
        return carry

    lax.fori_loop(0, GT // L, mol_body, 0)
    pltpu.sync_copy(molo_v, mol_out.at[pl.ds(gbase * MF, GT * MF)])

    def run_chunk(ch, buf):
        @plsc.parallel_loop(0, CH, step=L, unroll=4)
        def _(i):
            xv6 = x_v[pl.ds(ch * CH + i, L)] * NF
            for j in range(NF):
                vals = plsc.load_gather(tab_v, [xv6 + j] if j else [xv6])
                vals = (vals - means[j]) * rstds[j]
                plsc.store_scatter(buf, [i6 + (i * NF + j)], vals)

    bufs = (out_a, out_b)
    sems = (sem_a, sem_b)
    pending = [None, None]
    for ch in range(NCHUNK):
        b = ch % 2
        if pending[b] is not None:
            pending[b].wait()
        run_chunk(ch, bufs[b])
        dst = node_out.at[pl.ds((nbase + ch * CH) * NF, CH * NF)]
        pending[b] = pltpu.async_copy(bufs[b], dst, sems[b])
    pending[0].wait()
    pending[1].wait()


def kernel(x, num_atoms, radius, atom_properties_tensor,
           node_standardization_tensor, graph_standardization_tensor):
    node_flat, mol_flat = _featurize(
        x, num_atoms, radius,
        atom_properties_tensor.reshape(-1),
        node_standardization_tensor.reshape(-1),
        graph_standardization_tensor.reshape(-1))
    return node_flat.reshape(N_NODES, NF), mol_flat.reshape(N_GRAPHS, MF)


# speedup vs baseline: 34.1226x; 5.5164x over previous
import functools

import jax
import jax.numpy as jnp
from jax import lax
from jax.experimental import pallas as pl
from jax.experimental.pallas import tpu as pltpu
from jax.experimental.pallas import tpu_sc as plsc

N_NODES = 2097152
N_GRAPHS = 16384
N_ELEM = 100
NF = 6
MF = 2

NC, NS, L = 2, 16, 16
NW = NC * NS
NT = N_NODES // NW
CH = 4096
NCHUNK = NT // CH
GT = N_GRAPHS // NW

_mesh = plsc.VectorSubcoreMesh(
    core_axis_name="c", subcore_axis_name="s", num_cores=NC, num_subcores=NS
)


@functools.partial(
    pl.kernel,
    out_type=(
        jax.ShapeDtypeStruct((N_NODES * NF,), jnp.float32),
        jax.ShapeDtypeStruct((N_GRAPHS * MF,), jnp.float32),
    ),
    mesh=_mesh,
    compiler_params=pltpu.CompilerParams(needs_layout_passes=False, use_tc_tiling_on_sc=True),
    scratch_types=[
        pltpu.VMEM((NT,), jnp.int32),
        pltpu.VMEM((CH * NF,), jnp.float32),
        pltpu.VMEM((CH * NF,), jnp.float32),
        pltpu.VMEM((N_ELEM * NF,), jnp.float32),
        pltpu.VMEM((L,), jnp.float32),
        pltpu.VMEM((L,), jnp.float32),
        pltpu.VMEM((GT,), jnp.float32),
        pltpu.VMEM((GT,), jnp.float32),
        pltpu.VMEM((GT * MF,), jnp.float32),
        pltpu.SemaphoreType.DMA,
        pltpu.SemaphoreType.DMA,
    ],
)
def _featurize(x_hbm, na_hbm, rad_hbm, tab_hbm, nstd_hbm, gstd_hbm,
               node_out, mol_out,
               x_v, out_a, out_b, tab_v, nstd_v, gstd_v,
               na_v, rad_v, molo_v, sem_a, sem_b):
    wid = lax.axis_index("s") * NC + lax.axis_index("c")
    nbase = wid * NT
    gbase = wid * GT

    pltpu.sync_copy(x_hbm.at[pl.ds(nbase, NT)], x_v)
    pltpu.sync_copy(tab_hbm, tab_v)
    pltpu.sync_copy(nstd_hbm, nstd_v.at[pl.ds(0, NF * 2)])
    pltpu.sync_copy(gstd_hbm, gstd_v.at[pl.ds(0, MF * 2)])
    pltpu.sync_copy(na_hbm.at[pl.ds(gbase, GT)], na_v)
    pltpu.sync_copy(rad_hbm.at[pl.ds(gbase, GT)], rad_v)

    iota = lax.iota(jnp.int32, L)
    i6 = iota * NF
    i2 = iota * MF

    nv = nstd_v[pl.ds(0, L)]
    gv = gstd_v[pl.ds(0, L)]
    rnv = 1.0 / nv
    rgv = 1.0 / gv
    means = [jnp.full((L,), nv[2 * j]) for j in range(NF)]
    rstds = [jnp.full((L,), rnv[2 * j + 1]) for j in range(NF)]

    m_na = jnp.full((L,), gv[0])
    rs_na = jnp.full((L,), rgv[1])
    m_r = jnp.full((L,), gv[2])
    rs_r = jnp.full((L,), rgv[3])

    def mol_body(g, carry):
        na = (na_v[pl.ds(g * L, L)] - m_na) * rs_na
        rd = (rad_v[pl.ds(g * L, L)] - m_r) * rs_r
        molo_v[pl.ds(g * L, L)] = na
        molo_v[pl.ds(GT + g * L, L)] = rd
        return carry

    lax.fori_loop(0, GT // L, mol_body, 0)
    pltpu.sync_copy(molo_v.at[pl.ds(0, GT)], mol_out.at[pl.ds(gbase, GT)])
    pltpu.sync_copy(molo_v.at[pl.ds(GT, GT)],
                    mol_out.at[pl.ds(N_GRAPHS + gbase, GT)])

    def run_chunk(ch, buf):
        @plsc.parallel_loop(0, CH, step=L, unroll=4)
        def _(i):
            xv6 = x_v[pl.ds(ch * CH + i, L)] * NF
            for j in range(NF):
                vals = plsc.load_gather(tab_v, [xv6 + j] if j else [xv6])
                buf[pl.ds(j * CH + i, L)] = (vals - means[j]) * rstds[j]

    bufs = (out_a, out_b)
    sems = (sem_a, sem_b)
    pending = [None, None]
    for ch in range(NCHUNK):
        b = ch % 2
        if pending[b] is not None:
            for d in pending[b]:
                d.wait()
        run_chunk(ch, bufs[b])
        col0 = nbase + ch * CH
        pending[b] = [
            pltpu.async_copy(bufs[b].at[pl.ds(j * CH, CH)],
                             node_out.at[pl.ds(j * N_NODES + col0, CH)],
                             sems[b])
            for j in range(NF)
        ]
    for p in pending:
        for d in p:
            d.wait()


def kernel(x, num_atoms, radius, atom_properties_tensor,
           node_standardization_tensor, graph_standardization_tensor):
    node_flat, mol_flat = _featurize(
        x, num_atoms, radius,
        atom_properties_tensor.reshape(-1),
        node_standardization_tensor.reshape(-1),
        graph_standardization_tensor.reshape(-1))
    cols = [lax.slice(node_flat, (j * N_NODES,), ((j + 1) * N_NODES,))
            for j in range(NF)]
    return (jnp.stack(cols, axis=1),
            mol_flat.reshape(MF, N_GRAPHS).T)


# speedup vs baseline: 57.7646x; 1.6929x over previous
import functools

import jax
import jax.numpy as jnp
from jax import lax
from jax.experimental import pallas as pl
from jax.experimental.pallas import tpu as pltpu
from jax.experimental.pallas import tpu_sc as plsc

N_NODES = 2097152
N_GRAPHS = 16384
N_ELEM = 100
NF = 6
MF = 2

NC, NS, L = 2, 16, 16
NW = NC * NS
NT = N_NODES // NW
CH = 4096
NCHUNK = NT // CH
GT = N_GRAPHS // NW

_mesh = plsc.VectorSubcoreMesh(
    core_axis_name="c", subcore_axis_name="s", num_cores=NC, num_subcores=NS
)


@functools.partial(
    pl.kernel,
    out_type=(
        jax.ShapeDtypeStruct((N_NODES * NF,), jnp.float32),
        jax.ShapeDtypeStruct((N_GRAPHS * MF,), jnp.float32),
    ),
    mesh=_mesh,
    compiler_params=pltpu.CompilerParams(needs_layout_passes=False, use_tc_tiling_on_sc=True),
    scratch_types=[
        pltpu.VMEM((NT,), jnp.int32),
        pltpu.VMEM((CH * NF,), jnp.float32),
        pltpu.VMEM((CH * NF,), jnp.float32),
        pltpu.VMEM((N_ELEM * NF,), jnp.float32),
        pltpu.VMEM((L,), jnp.float32),
        pltpu.VMEM((L,), jnp.float32),
        pltpu.VMEM((GT,), jnp.float32),
        pltpu.VMEM((GT,), jnp.float32),
        pltpu.VMEM((GT * MF,), jnp.float32),
        pltpu.SemaphoreType.DMA,
        pltpu.SemaphoreType.DMA,
    ],
)
def _featurize(x_hbm, na_hbm, rad_hbm, tab_hbm, nstd_hbm, gstd_hbm,
               node_out, mol_out,
               x_v, out_a, out_b, tab_v, nstd_v, gstd_v,
               na_v, rad_v, molo_v, sem_a, sem_b):
    wid = lax.axis_index("s") * NC + lax.axis_index("c")
    nbase = wid * NT
    gbase = wid * GT

    pltpu.sync_copy(x_hbm.at[pl.ds(nbase, NT)], x_v)
    pltpu.sync_copy(tab_hbm, tab_v)
    pltpu.sync_copy(nstd_hbm, nstd_v.at[pl.ds(0, NF * 2)])
    pltpu.sync_copy(gstd_hbm, gstd_v.at[pl.ds(0, MF * 2)])
    pltpu.sync_copy(na_hbm.at[pl.ds(gbase, GT)], na_v)
    pltpu.sync_copy(rad_hbm.at[pl.ds(gbase, GT)], rad_v)

    iota = lax.iota(jnp.int32, L)
    i6 = iota * NF
    i2 = iota * MF

    nv = nstd_v[pl.ds(0, L)]
    gv = gstd_v[pl.ds(0, L)]
    rnv = 1.0 / nv
    rgv = 1.0 / gv
    means = [jnp.full((L,), nv[2 * j]) for j in range(NF)]
    rstds = [jnp.full((L,), rnv[2 * j + 1]) for j in range(NF)]

    m_na = jnp.full((L,), gv[0])
    rs_na = jnp.full((L,), rgv[1])
    m_r = jnp.full((L,), gv[2])
    rs_r = jnp.full((L,), rgv[3])

    def mol_body(g, carry):
        na = (na_v[pl.ds(g * L, L)] - m_na) * rs_na
        rd = (rad_v[pl.ds(g * L, L)] - m_r) * rs_r
        molo_v[pl.ds(g * L, L)] = na
        molo_v[pl.ds(GT + g * L, L)] = rd
        return carry

    lax.fori_loop(0, GT // L, mol_body, 0)
    pltpu.sync_copy(molo_v.at[pl.ds(0, GT)], mol_out.at[pl.ds(gbase, GT)])
    pltpu.sync_copy(molo_v.at[pl.ds(GT, GT)],
                    mol_out.at[pl.ds(N_GRAPHS + gbase, GT)])

    def run_chunk(ch, buf):
        @plsc.parallel_loop(0, CH, step=L, unroll=4)
        def _(i):
            xv6 = x_v[pl.ds(ch * CH + i, L)] * NF
            for j in range(NF):
                vals = plsc.load_gather(tab_v, [xv6 + j] if j else [xv6])
                buf[pl.ds(j * CH + i, L)] = (vals - means[j]) * rstds[j]

    bufs = (out_a, out_b)
    sems = (sem_a, sem_b)
    pending = [None, None]
    for ch in range(NCHUNK):
        b = ch % 2
        if pending[b] is not None:
            for d in pending[b]:
                d.wait()
        run_chunk(ch, bufs[b])
        col0 = nbase + ch * CH
        pending[b] = [
            pltpu.async_copy(bufs[b].at[pl.ds(j * CH, CH)],
                             node_out.at[pl.ds(j * N_NODES + col0, CH)],
                             sems[b])
            for j in range(NF)
        ]
    for p in pending:
        for d in p:
            d.wait()


_BL = 16384


def _interleave_body(*refs):
    out_ref = refs[-1]
    out_ref[...] = jnp.stack([refs[j][...] for j in range(NF)], axis=0)


def _interleave(node_flat):
    nb = N_NODES // _BL
    in_specs = [
        pl.BlockSpec((_BL,), lambda i, j=j: (j * nb + i,)) for j in range(NF)
    ]
    return pl.pallas_call(
        _interleave_body,
        grid=(nb,),
        in_specs=in_specs,
        out_specs=pl.BlockSpec((NF, _BL), lambda i: (0, i)),
        out_shape=jax.ShapeDtypeStruct((NF, N_NODES), jnp.float32),
    )(*([node_flat] * NF))


def kernel(x, num_atoms, radius, atom_properties_tensor,
           node_standardization_tensor, graph_standardization_tensor):
    node_flat, mol_flat = _featurize(
        x, num_atoms, radius,
        atom_properties_tensor.reshape(-1),
        node_standardization_tensor.reshape(-1),
        graph_standardization_tensor.reshape(-1))
    return (_interleave(node_flat).T,
            mol_flat.reshape(MF, N_GRAPHS).T)


# speedup vs baseline: 138.7390x; 2.4018x over previous
import functools

import jax
import jax.numpy as jnp
from jax import lax
from jax.experimental import pallas as pl
from jax.experimental.pallas import tpu as pltpu
from jax.experimental.pallas import tpu_sc as plsc

N_NODES = 2097152
N_GRAPHS = 16384
N_ELEM = 100
NF = 6
MF = 2

NC, NS, L = 2, 16, 16
NW = NC * NS
NT = N_NODES // NW
CH = 2048
NCHUNK = NT // CH
GT = N_GRAPHS // NW

_mesh = plsc.VectorSubcoreMesh(
    core_axis_name="c", subcore_axis_name="s", num_cores=NC, num_subcores=NS
)


@functools.partial(
    pl.kernel,
    out_type=(
        jax.ShapeDtypeStruct((NF, N_NODES), jnp.float32),
        jax.ShapeDtypeStruct((N_GRAPHS * MF,), jnp.float32),
    ),
    mesh=_mesh,
    compiler_params=pltpu.CompilerParams(needs_layout_passes=False, use_tc_tiling_on_sc=True),
    scratch_types=[
        pltpu.VMEM((NT,), jnp.int32),
        pltpu.VMEM((NF, CH), jnp.float32),
        pltpu.VMEM((NF, CH), jnp.float32),
        pltpu.VMEM((N_ELEM * NF,), jnp.float32),
        pltpu.VMEM((L,), jnp.float32),
        pltpu.VMEM((L,), jnp.float32),
        pltpu.VMEM((GT,), jnp.float32),
        pltpu.VMEM((GT,), jnp.float32),
        pltpu.VMEM((GT * MF,), jnp.float32),
        pltpu.SemaphoreType.DMA,
        pltpu.SemaphoreType.DMA,
    ],
)
def _featurize(x_hbm, na_hbm, rad_hbm, tab_hbm, nstd_hbm, gstd_hbm,
               node_out, mol_out,
               x_v, out_a, out_b, tab_v, nstd_v, gstd_v,
               na_v, rad_v, molo_v, sem_a, sem_b):
    wid = lax.axis_index("s") * NC + lax.axis_index("c")
    nbase = wid * NT
    gbase = wid * GT

    pltpu.sync_copy(x_hbm.at[pl.ds(nbase, NT)], x_v)
    pltpu.sync_copy(tab_hbm, tab_v)
    pltpu.sync_copy(nstd_hbm, nstd_v.at[pl.ds(0, NF * 2)])
    pltpu.sync_copy(gstd_hbm, gstd_v.at[pl.ds(0, MF * 2)])
    pltpu.sync_copy(na_hbm.at[pl.ds(gbase, GT)], na_v)
    pltpu.sync_copy(rad_hbm.at[pl.ds(gbase, GT)], rad_v)

    iota = lax.iota(jnp.int32, L)
    i6 = iota * NF
    i2 = iota * MF

    nv = nstd_v[pl.ds(0, L)]
    gv = gstd_v[pl.ds(0, L)]
    rnv = 1.0 / nv
    rgv = 1.0 / gv
    means = [jnp.full((L,), nv[2 * j]) for j in range(NF)]
    rstds = [jnp.full((L,), rnv[2 * j + 1]) for j in range(NF)]

    m_na = jnp.full((L,), gv[0])
    rs_na = jnp.full((L,), rgv[1])
    m_r = jnp.full((L,), gv[2])
    rs_r = jnp.full((L,), rgv[3])

    def mol_body(g, carry):
        na = (na_v[pl.ds(g * L, L)] - m_na) * rs_na
        rd = (rad_v[pl.ds(g * L, L)] - m_r) * rs_r
        molo_v[pl.ds(g * L, L)] = na
        molo_v[pl.ds(GT + g * L, L)] = rd
        return carry

    lax.fori_loop(0, GT // L, mol_body, 0)
    pltpu.sync_copy(molo_v.at[pl.ds(0, GT)], mol_out.at[pl.ds(gbase, GT)])
    pltpu.sync_copy(molo_v.at[pl.ds(GT, GT)],
                    mol_out.at[pl.ds(N_GRAPHS + gbase, GT)])

    def run_chunk(ch, buf):
        @plsc.parallel_loop(0, CH, step=L, unroll=4)
        def _(i):
            xv6 = x_v[pl.ds(ch * CH + i, L)] * NF
            for j in range(NF):
                vals = plsc.load_gather(tab_v, [xv6 + j] if j else [xv6])
                buf[j, pl.ds(i, L)] = (vals - means[j]) * rstds[j]

    bufs = (out_a, out_b)
    sems = (sem_a, sem_b)
    pending = [None, None]
    for ch in range(NCHUNK):
        b = ch % 2
        if pending[b] is not None:
            for d in pending[b]:
                d.wait()
        run_chunk(ch, bufs[b])
        col0 = nbase + ch * CH
        pending[b] = [
            pltpu.async_copy(bufs[b],
                             node_out.at[:, pl.ds(col0, CH)],
                             sems[b])
        ]
    for p in pending:
        for d in p:
            d.wait()


def kernel(x, num_atoms, radius, atom_properties_tensor,
           node_standardization_tensor, graph_standardization_tensor):
    node_cm, mol_flat = _featurize(
        x, num_atoms, radius,
        atom_properties_tensor.reshape(-1),
        node_standardization_tensor.reshape(-1),
        graph_standardization_tensor.reshape(-1))
    return (node_cm.T,
            mol_flat.reshape(MF, N_GRAPHS).T)


# speedup vs baseline: 150.4642x; 1.0845x over previous
import functools

import jax
import jax.numpy as jnp
from jax import lax
from jax.experimental import pallas as pl
from jax.experimental.pallas import tpu as pltpu
from jax.experimental.pallas import tpu_sc as plsc

N_NODES = 2097152
N_GRAPHS = 16384
N_ELEM = 100
NF = 6
MF = 2

NC, NS, L = 2, 16, 16
NW = NC * NS
NT = N_NODES // NW
CH = 2048
NCHUNK = NT // CH
GT = N_GRAPHS // NW

_mesh = plsc.VectorSubcoreMesh(
    core_axis_name="c", subcore_axis_name="s", num_cores=NC, num_subcores=NS
)


@functools.partial(
    pl.kernel,
    out_type=(
        jax.ShapeDtypeStruct((NF, N_NODES), jnp.float32),
        jax.ShapeDtypeStruct((N_GRAPHS * MF,), jnp.float32),
    ),
    mesh=_mesh,
    compiler_params=pltpu.CompilerParams(needs_layout_passes=False, use_tc_tiling_on_sc=True),
    scratch_types=[
        pltpu.VMEM((NT,), jnp.int32),
        pltpu.VMEM((NF, CH), jnp.float32),
        pltpu.VMEM((NF, CH), jnp.float32),
        pltpu.VMEM((NF * 112,), jnp.float32),
        pltpu.VMEM((L,), jnp.float32),
        pltpu.VMEM((L,), jnp.float32),
        pltpu.VMEM((GT,), jnp.float32),
        pltpu.VMEM((GT,), jnp.float32),
        pltpu.VMEM((GT * MF,), jnp.float32),
        pltpu.SemaphoreType.DMA,
        pltpu.SemaphoreType.DMA,
    ],
)
def _featurize(x_hbm, na_hbm, rad_hbm, tab_hbm, nstd_hbm, gstd_hbm,
               node_out, mol_out,
               x_v, out_a, out_b, tab_v, nstd_v, gstd_v,
               na_v, rad_v, molo_v, sem_a, sem_b):
    wid = lax.axis_index("s") * NC + lax.axis_index("c")
    nbase = wid * NT
    gbase = wid * GT

    pltpu.sync_copy(x_hbm.at[pl.ds(nbase, NT)], x_v)
    pltpu.sync_copy(tab_hbm, tab_v)
    pltpu.sync_copy(nstd_hbm, nstd_v.at[pl.ds(0, NF * 2)])
    pltpu.sync_copy(gstd_hbm, gstd_v.at[pl.ds(0, MF * 2)])
    pltpu.sync_copy(na_hbm.at[pl.ds(gbase, GT)], na_v)
    pltpu.sync_copy(rad_hbm.at[pl.ds(gbase, GT)], rad_v)

    iota = lax.iota(jnp.int32, L)
    i6 = iota * NF
    i2 = iota * MF

    nv = nstd_v[pl.ds(0, L)]
    gv = gstd_v[pl.ds(0, L)]
    rnv = 1.0 / nv
    rgv = 1.0 / gv
    means = [jnp.full((L,), nv[2 * j]) for j in range(NF)]
    rstds = [jnp.full((L,), rnv[2 * j + 1]) for j in range(NF)]

    m_na = jnp.full((L,), gv[0])
    rs_na = jnp.full((L,), rgv[1])
    m_r = jnp.full((L,), gv[2])
    rs_r = jnp.full((L,), rgv[3])

    for j in range(NF):
        for g in range(7):
            sl = pl.ds(j * 112 + g * L, L)
            tab_v[sl] = (tab_v[sl] - means[j]) * rstds[j]

    def mol_body(g, carry):
        na = (na_v[pl.ds(g * L, L)] - m_na) * rs_na
        rd = (rad_v[pl.ds(g * L, L)] - m_r) * rs_r
        molo_v[pl.ds(g * L, L)] = na
        molo_v[pl.ds(GT + g * L, L)] = rd
        return carry

    lax.fori_loop(0, GT // L, mol_body, 0)
    pltpu.sync_copy(molo_v.at[pl.ds(0, GT)], mol_out.at[pl.ds(gbase, GT)])
    pltpu.sync_copy(molo_v.at[pl.ds(GT, GT)],
                    mol_out.at[pl.ds(N_GRAPHS + gbase, GT)])

    def run_chunk(ch, buf):
        @plsc.parallel_loop(0, CH, step=L, unroll=8)
        def _(i):
            xv = x_v[pl.ds(ch * CH + i, L)]
            for j in range(NF):
                buf[j, pl.ds(i, L)] = plsc.load_gather(
                    tab_v, [xv + 112 * j] if j else [xv])

    bufs = (out_a, out_b)
    sems = (sem_a, sem_b)
    pending = [None, None]
    for ch in range(NCHUNK):
        b = ch % 2
        if pending[b] is not None:
            for d in pending[b]:
                d.wait()
        run_chunk(ch, bufs[b])
        col0 = nbase + ch * CH
        pending[b] = [
            pltpu.async_copy(bufs[b],
                             node_out.at[:, pl.ds(col0, CH)],
                             sems[b])
        ]
    for p in pending:
        for d in p:
            d.wait()


def kernel(x, num_atoms, radius, atom_properties_tensor,
           node_standardization_tensor, graph_standardization_tensor):
    node_cm, mol_flat = _featurize(
        x, num_atoms, radius,
        jnp.pad(atom_properties_tensor.T, ((0, 0), (0, 112 - N_ELEM))).reshape(-1),
        node_standardization_tensor.reshape(-1),
        graph_standardization_tensor.reshape(-1))
    return (node_cm.T,
            mol_flat.reshape(MF, N_GRAPHS).T)
